# Initial kernel scaffold; baseline (speedup 1.0000x reference)
import jax
import jax.numpy as jnp
from jax.experimental import pallas as pl


def kernel(x, pos, W, b, batch):
    raise NotImplementedError("write your pallas kernel here")


# speedup vs baseline: 1.4939x; 1.4939x over previous
import functools

import jax
import jax.numpy as jnp
from jax import lax
from jax.experimental import pallas as pl
from jax.experimental.pallas import tpu as pltpu
from jax.experimental.pallas import tpu_sc as plsc

N = 50000
D_IN = 128
D_OUT = 128
B = 64
K = 32
DP = 256
BLK = 1024
NPAD = 52224
CH = 2048
NEG_INF = float("-inf")


def _score_count_body(xp_ref, w_ref, batch_ref, s_ref, cnt_ref):
    i = pl.program_id(0)
    s_ref[...] = jnp.dot(xp_ref[...], w_ref[...],
                         preferred_element_type=jnp.float32)
    gids = lax.broadcasted_iota(jnp.int32, (1, B), 1)
    part = jnp.sum((batch_ref[...] == gids).astype(jnp.int32), axis=0,
                   keepdims=True)
    row0 = (lax.broadcasted_iota(jnp.int32, (8, B), 0) == 0).astype(jnp.int32)
    upd = part * row0

    @pl.when(i == 0)
    def _():
        cnt_ref[...] = jnp.zeros_like(cnt_ref)

    cnt_ref[...] += upd


def _topk_gather_body(scores_hbm, counts_hbm, xp_hbm, sel_hbm, mask_hbm,
                      cnt_v, seg_v, idx_v, rows_v, mask_v, sem):
    wid = lax.axis_index("s") * 2 + lax.axis_index("c")
    pltpu.sync_copy(counts_hbm, cnt_v.at[pl.ds(0, B)])
    iota16 = lax.iota(jnp.int32, 16)

    for t in range(2):
        g = wid + 32 * t
        seg_len = cnt_v[pl.ds(g, 16)][0]
        startf = jnp.float32(0)
        for j in range(4):
            v = cnt_v[pl.ds(j * 16, 16)].astype(jnp.float32)
            m = (j * 16 + iota16) < g
            startf = startf + jnp.sum(jnp.where(m, v, jnp.float32(0)))
        start = startf.astype(jnp.int32)
        end = start + seg_len
        astart = (start // 8) * 8
        total = end - astart
        nch = (total + CH - 1) // CH
        nv = (total + 15) // 16

        def stage(c, _):
            pltpu.sync_copy(scores_hbm.at[pl.ds(astart + c * CH, CH)],
                            seg_v.at[pl.ds(c * CH, CH)])
            return 0

        lax.fori_loop(0, nch, stage, 0)

        def maskpass(j, _):
            v = seg_v[pl.ds(j * 16, 16)]
            gx = astart + j * 16 + iota16
            ok = (gx >= start) & (gx < end)
            seg_v[pl.ds(j * 16, 16)] = jnp.where(ok, v, NEG_INF)
            return 0

        lax.fori_loop(0, nv, maskpass, 0)

        def round_body(r, carry):
            lo, hi = carry

            def maxpass(j, acc):
                return jnp.maximum(acc, seg_v[pl.ds(j * 16, 16)])

            acc = lax.fori_loop(0, nv, maxpass,
                                jnp.full((16,), NEG_INF, jnp.float32))
            mx = jnp.max(acc)

            def findpass(j, carry2):
                found, pos = carry2
                v = seg_v[pl.ds(j * 16, 16)]
                eq = v == mx
                anyeq = jnp.max(eq.astype(jnp.int32))
                lane = jnp.min(jnp.where(eq, iota16, jnp.int32(9999)))
                hit = (found == 0) & (anyeq == 1)
                pos = jnp.where(hit, j * 16 + lane, pos)
                return (found | anyeq, pos)

            _, pos = lax.fori_loop(0, nv, findpass,
                                   (jnp.int32(0), jnp.int32(0)))
            gidx = astart + pos
            lo = jnp.where(iota16 == r, gidx, lo)
            hi = jnp.where(iota16 == (r - 16), gidx, hi)
            jv = pos // 16
            v = seg_v[pl.ds(jv * 16, 16)]
            seg_v[pl.ds(jv * 16, 16)] = jnp.where(iota16 == (pos % 16),
                                                  NEG_INF, v)
            return (lo, hi)

        zero16 = jnp.zeros((16,), jnp.int32)
        idx_lo, idx_hi = lax.fori_loop(0, K, round_body, (zero16, zero16))
        idx_v[pl.ds(0, 16)] = idx_lo
        idx_v[pl.ds(16, 16)] = idx_hi

        mask_v[pl.ds(0, 16)] = (iota16 < seg_len).astype(jnp.float32)
        mask_v[pl.ds(16, 16)] = ((iota16 + 16) < seg_len).astype(jnp.float32)

        pltpu.async_copy(xp_hbm.at[idx_v], rows_v, sem).wait()
        pltpu.sync_copy(rows_v, sel_hbm.at[pl.ds(g * K, K)])
        pltpu.sync_copy(mask_v, mask_hbm.at[pl.ds(g * K, K)])


def _project_body(sel_ref, w_ref, b_ref, mask_ref, out_ref):
    h = jnp.dot(sel_ref[...], w_ref[...], preferred_element_type=jnp.float32)
    out_ref[...] = (h + b_ref[...]) * mask_ref[...]


def kernel(x, pos, W, b, batch):
    f32 = jnp.float32
    xp = jnp.zeros((NPAD, DP), f32)
    xp = xp.at[:N, :D_IN].set(x.astype(f32))
    xp = xp.at[:N, D_IN:D_IN + 3].set(pos.astype(f32))
    Wp = jnp.zeros((DP, D_OUT), f32).at[:D_IN + 3, :].set(W.astype(f32))
    wcol = Wp[:, D_OUT - 1:D_OUT]
    batch_p = jnp.full((NPAD, 1), B, jnp.int32)
    batch_p = batch_p.at[:N, 0].set(batch.astype(jnp.int32))

    scores, counts2d = pl.pallas_call(
        _score_count_body,
        grid=(NPAD // BLK,),
        in_specs=[
            pl.BlockSpec((BLK, DP), lambda i: (i, 0)),
            pl.BlockSpec((DP, 1), lambda i: (0, 0)),
            pl.BlockSpec((BLK, 1), lambda i: (i, 0)),
        ],
        out_specs=[
            pl.BlockSpec((BLK, 1), lambda i: (i, 0)),
            pl.BlockSpec((8, B), lambda i: (0, 0)),
        ],
        out_shape=[
            jax.ShapeDtypeStruct((NPAD, 1), f32),
            jax.ShapeDtypeStruct((8, B), jnp.int32),
        ],
    )(xp, wcol, batch_p)

    mesh = plsc.VectorSubcoreMesh(core_axis_name="c", subcore_axis_name="s")
    topk = functools.partial(
        pl.kernel,
        mesh=mesh,
        compiler_params=pltpu.CompilerParams(needs_layout_passes=False),
        out_type=[
            jax.ShapeDtypeStruct((B * K, DP), f32),
            jax.ShapeDtypeStruct((B * K,), f32),
        ],
        scratch_types=[
            pltpu.VMEM((B + 16,), jnp.int32),
            pltpu.VMEM((28 * CH,), f32),
            pltpu.VMEM((K,), jnp.int32),
            pltpu.VMEM((K, DP), f32),
            pltpu.VMEM((K,), f32),
            pltpu.SemaphoreType.DMA,
        ],
    )(_topk_gather_body)
    sel, maskv = topk(scores.reshape(NPAD), counts2d[0], xp)

    pooled2d = pl.pallas_call(
        _project_body,
        out_shape=jax.ShapeDtypeStruct((B * K, D_OUT), f32),
    )(sel, Wp, b.astype(f32).reshape(1, D_OUT), maskv.reshape(B * K, 1))

    pooled = pooled2d.reshape(B, K * D_OUT)
    pos_out = jnp.zeros((B, 3), f32)
    batch_out = jnp.arange(B, dtype=jnp.int64)
    return (pooled, pos_out, batch_out)


# speedup vs baseline: 2.7808x; 1.8615x over previous
import functools

import jax
import jax.numpy as jnp
from jax import lax
from jax.experimental import pallas as pl
from jax.experimental.pallas import tpu as pltpu
from jax.experimental.pallas import tpu_sc as plsc

N = 50000
D_IN = 128
D_OUT = 128
B = 64
K = 32
BLK = 1024
NPAD = 52224
CH = 2048
NEG_INF = float("-inf")


def _score_count_body(x_ref, pos_ref, wx_ref, wp_ref, batch_ref, s_ref,
                      cnt_ref):
    i = pl.program_id(0)
    s_ref[...] = (
        jnp.dot(x_ref[...], wx_ref[...], preferred_element_type=jnp.float32)
        + jnp.dot(pos_ref[...], wp_ref[...],
                  preferred_element_type=jnp.float32))
    gids = lax.broadcasted_iota(jnp.int32, (1, B), 1)
    part = jnp.sum((batch_ref[...] == gids).astype(jnp.int32), axis=0,
                   keepdims=True)
    row0 = (lax.broadcasted_iota(jnp.int32, (8, B), 0) == 0).astype(jnp.int32)
    upd = part * row0

    @pl.when(i == 0)
    def _():
        cnt_ref[...] = jnp.zeros_like(cnt_ref)

    cnt_ref[...] += upd


def _topk_gather_body(scores_hbm, counts_hbm, x_hbm, posp_hbm,
                      selx_hbm, selp_hbm, mask_hbm,
                      cnt_v, seg_v, idx_v, rowsx_v, rowsp_v, mask_v, sem):
    wid = lax.axis_index("s") * 2 + lax.axis_index("c")
    pltpu.sync_copy(counts_hbm, cnt_v.at[pl.ds(0, B)])
    iota16 = lax.iota(jnp.int32, 16)

    for t in range(2):
        g = wid + 32 * t
        seg_len = cnt_v[pl.ds(g, 16)][0]
        startf = jnp.float32(0)
        for j in range(4):
            v = cnt_v[pl.ds(j * 16, 16)].astype(jnp.float32)
            m = (j * 16 + iota16) < g
            startf = startf + jnp.sum(jnp.where(m, v, jnp.float32(0)))
        start = startf.astype(jnp.int32)
        end = start + seg_len
        astart = (start // 8) * 8
        total = end - astart
        nch = (total + CH - 1) // CH
        nv = (total + 15) // 16

        def stage(c, _):
            pltpu.sync_copy(scores_hbm.at[pl.ds(astart + c * CH, CH)],
                            seg_v.at[pl.ds(c * CH, CH)])
            return 0

        lax.fori_loop(0, nch, stage, 0)

        def maskpass(j, _):
            v = seg_v[pl.ds(j * 16, 16)]
            gx = astart + j * 16 + iota16
            ok = (gx >= start) & (gx < end)
            seg_v[pl.ds(j * 16, 16)] = jnp.where(ok, v, NEG_INF)
            return 0

        lax.fori_loop(0, nv, maskpass, 0)

        def round_body(r, carry):
            lo, hi = carry

            def maxpass(j, acc):
                return jnp.maximum(acc, seg_v[pl.ds(j * 16, 16)])

            acc = lax.fori_loop(0, nv, maxpass,
                                jnp.full((16,), NEG_INF, jnp.float32))
            mx = jnp.max(acc)

            def findpass(j, carry2):
                found, pos = carry2
                v = seg_v[pl.ds(j * 16, 16)]
                eq = v == mx
                anyeq = jnp.max(eq.astype(jnp.int32))
                lane = jnp.min(jnp.where(eq, iota16, jnp.int32(9999)))
                hit = (found == 0) & (anyeq == 1)
                pos = jnp.where(hit, j * 16 + lane, pos)
                return (found | anyeq, pos)

            _, pos = lax.fori_loop(0, nv, findpass,
                                   (jnp.int32(0), jnp.int32(0)))
            gidx = astart + pos
            lo = jnp.where(iota16 == r, gidx, lo)
            hi = jnp.where(iota16 == (r - 16), gidx, hi)
            jv = pos // 16
            v = seg_v[pl.ds(jv * 16, 16)]
            seg_v[pl.ds(jv * 16, 16)] = jnp.where(iota16 == (pos % 16),
                                                  NEG_INF, v)
            return (lo, hi)

        zero16 = jnp.zeros((16,), jnp.int32)
        idx_lo, idx_hi = lax.fori_loop(0, K, round_body, (zero16, zero16))
        idx_v[pl.ds(0, 16)] = idx_lo
        idx_v[pl.ds(16, 16)] = idx_hi

        mask_v[pl.ds(0, 16)] = (iota16 < seg_len).astype(jnp.float32)
        mask_v[pl.ds(16, 16)] = ((iota16 + 16) < seg_len).astype(jnp.float32)

        cx = pltpu.async_copy(x_hbm.at[idx_v], rowsx_v, sem)
        cp = pltpu.async_copy(posp_hbm.at[idx_v], rowsp_v, sem)
        cx.wait()
        cp.wait()
        pltpu.sync_copy(rowsx_v, selx_hbm.at[pl.ds(g * K, K)])
        pltpu.sync_copy(rowsp_v, selp_hbm.at[pl.ds(g * K, K)])
        pltpu.sync_copy(mask_v, mask_hbm.at[pl.ds(g * K, K)])


def _project_body(selx_ref, selp_ref, wx_ref, wp_ref, b_ref, mask_ref,
                  out_ref):
    h = (jnp.dot(selx_ref[...], wx_ref[...],
                 preferred_element_type=jnp.float32)
         + jnp.dot(selp_ref[...], wp_ref[...],
                   preferred_element_type=jnp.float32))
    out_ref[...] = (h + b_ref[...]) * mask_ref[...]


def kernel(x, pos, W, b, batch):
    f32 = jnp.float32
    x = x.astype(f32)
    pos = pos.astype(f32)
    W = W.astype(f32)
    posp = jnp.pad(pos, ((0, 0), (0, D_IN - 3)))
    wx = W[:D_IN, D_OUT - 1:D_OUT]
    wp = W[D_IN:D_IN + 3, D_OUT - 1:D_OUT]
    Wx = W[:D_IN, :]
    Wp = jnp.zeros((D_IN, D_OUT), f32).at[:3, :].set(W[D_IN:D_IN + 3, :])
    batch_p = jnp.full((NPAD, 1), B, jnp.int32)
    batch_p = batch_p.at[:N, 0].set(batch.astype(jnp.int32))

    nfull = (N - 1) // BLK

    scores, counts2d = pl.pallas_call(
        _score_count_body,
        grid=(NPAD // BLK,),
        in_specs=[
            pl.BlockSpec((BLK, D_IN), lambda i: (jnp.minimum(i, nfull), 0)),
            pl.BlockSpec((BLK, 3), lambda i: (jnp.minimum(i, nfull), 0)),
            pl.BlockSpec((D_IN, 1), lambda i: (0, 0)),
            pl.BlockSpec((3, 1), lambda i: (0, 0)),
            pl.BlockSpec((BLK, 1), lambda i: (i, 0)),
        ],
        out_specs=[
            pl.BlockSpec((BLK, 1), lambda i: (i, 0)),
            pl.BlockSpec((8, B), lambda i: (0, 0)),
        ],
        out_shape=[
            jax.ShapeDtypeStruct((NPAD, 1), f32),
            jax.ShapeDtypeStruct((8, B), jnp.int32),
        ],
    )(x, pos, wx, wp, batch_p)

    mesh = plsc.VectorSubcoreMesh(core_axis_name="c", subcore_axis_name="s")
    topk = functools.partial(
        pl.kernel,
        mesh=mesh,
        compiler_params=pltpu.CompilerParams(needs_layout_passes=False),
        out_type=[
            jax.ShapeDtypeStruct((B * K, D_IN), f32),
            jax.ShapeDtypeStruct((B * K, D_IN), f32),
            jax.ShapeDtypeStruct((B * K,), f32),
        ],
        scratch_types=[
            pltpu.VMEM((B + 16,), jnp.int32),
            pltpu.VMEM((28 * CH,), f32),
            pltpu.VMEM((K,), jnp.int32),
            pltpu.VMEM((K, D_IN), f32),
            pltpu.VMEM((K, D_IN), f32),
            pltpu.VMEM((K,), f32),
            pltpu.SemaphoreType.DMA,
        ],
    )(_topk_gather_body)
    selx, selp, maskv = topk(scores.reshape(NPAD), counts2d[0], x, posp)

    pooled2d = pl.pallas_call(
        _project_body,
        out_shape=jax.ShapeDtypeStruct((B * K, D_OUT), f32),
    )(selx, selp, Wx, Wp, b.astype(f32).reshape(1, D_OUT),
      maskv.reshape(B * K, 1))

    pooled = pooled2d.reshape(B, K * D_OUT)
    pos_out = jnp.zeros((B, 3), f32)
    batch_out = jnp.arange(B, dtype=jnp.int64)
    return (pooled, pos_out, batch_out)


# speedup vs baseline: 2.8125x; 1.0114x over previous
import functools

import jax
import jax.numpy as jnp
from jax import lax
from jax.experimental import pallas as pl
from jax.experimental.pallas import tpu as pltpu
from jax.experimental.pallas import tpu_sc as plsc

N = 50000
D_IN = 128
D_OUT = 128
B = 64
K = 32
BLK = 1024
NPAD = 52224
CH = 2048
NEG_INF = float("-inf")


def _score_count_body(x_ref, pos_ref, wx_ref, wp_ref, batch_ref, s_ref,
                      cnt_ref):
    i = pl.program_id(0)
    s_ref[...] = (
        jnp.dot(x_ref[...], wx_ref[...], preferred_element_type=jnp.float32)
        + jnp.dot(pos_ref[...], wp_ref[...],
                  preferred_element_type=jnp.float32))
    gids = lax.broadcasted_iota(jnp.int32, (1, B), 1)
    part = jnp.sum((batch_ref[...] == gids).astype(jnp.int32), axis=0,
                   keepdims=True)
    row0 = (lax.broadcasted_iota(jnp.int32, (8, B), 0) == 0).astype(jnp.int32)
    upd = part * row0

    @pl.when(i == 0)
    def _():
        cnt_ref[...] = jnp.zeros_like(cnt_ref)

    cnt_ref[...] += upd


def _topk_gather_body(scores_hbm, counts_hbm, x_hbm, posp_hbm,
                      selx_hbm, selp_hbm, mask_hbm,
                      cnt_v, seg_v, idx_v, rowsx_v, rowsp_v, mask_v, sem):
    wid = lax.axis_index("s") * 2 + lax.axis_index("c")
    pltpu.sync_copy(counts_hbm, cnt_v.at[pl.ds(0, B)])
    iota16 = lax.iota(jnp.int32, 16)

    for t in range(2):
        g = wid + 32 * t
        seg_len = cnt_v[pl.ds(g, 16)][0]
        startf = jnp.float32(0)
        for j in range(4):
            v = cnt_v[pl.ds(j * 16, 16)].astype(jnp.float32)
            m = (j * 16 + iota16) < g
            startf = startf + jnp.sum(jnp.where(m, v, jnp.float32(0)))
        start = startf.astype(jnp.int32)
        end = start + seg_len
        astart = (start // 8) * 8
        total = end - astart
        nch = (total + CH - 1) // CH
        nv = (total + 15) // 16

        def stage(c, _):
            pltpu.sync_copy(scores_hbm.at[pl.ds(astart + c * CH, CH)],
                            seg_v.at[pl.ds(c * CH, CH)])
            return 0

        lax.fori_loop(0, nch, stage, 0)

        def maskpass(j, _):
            v = seg_v[pl.ds(j * 16, 16)]
            gx = astart + j * 16 + iota16
            ok = (gx >= start) & (gx < end)
            seg_v[pl.ds(j * 16, 16)] = jnp.where(ok, v, NEG_INF)
            return 0

        lax.fori_loop(0, nv, maskpass, 0)

        def round_body(r, carry):
            lo, hi = carry

            def maxpass(j, acc):
                return jnp.maximum(acc, seg_v[pl.ds(j * 16, 16)])

            acc = lax.fori_loop(0, nv, maxpass,
                                jnp.full((16,), NEG_INF, jnp.float32))
            mx = jnp.max(acc)

            def findpass(j, accmin):
                v = seg_v[pl.ds(j * 16, 16)]
                cand = jnp.where(v == mx, j * 16 + iota16,
                                 jnp.int32(1 << 30))
                return jnp.minimum(accmin, cand)

            accmin = lax.fori_loop(0, nv, findpass,
                                   jnp.full((16,), 1 << 30, jnp.int32))
            pos = jnp.minimum(jnp.min(accmin),
                              jnp.maximum(total - 1, jnp.int32(0)))
            gidx = jnp.minimum(astart + pos, jnp.int32(N - 1))
            lo = jnp.where(iota16 == r, gidx, lo)
            hi = jnp.where(iota16 == (r - 16), gidx, hi)
            jv = pos // 16
            v = seg_v[pl.ds(jv * 16, 16)]
            seg_v[pl.ds(jv * 16, 16)] = jnp.where(iota16 == (pos % 16),
                                                  NEG_INF, v)
            return (lo, hi)

        zero16 = jnp.zeros((16,), jnp.int32)
        idx_lo, idx_hi = lax.fori_loop(0, K, round_body, (zero16, zero16))
        idx_v[pl.ds(0, 16)] = idx_lo
        idx_v[pl.ds(16, 16)] = idx_hi

        mask_v[pl.ds(0, 16)] = (iota16 < seg_len).astype(jnp.float32)
        mask_v[pl.ds(16, 16)] = ((iota16 + 16) < seg_len).astype(jnp.float32)

        cx = pltpu.async_copy(x_hbm.at[idx_v], rowsx_v, sem)
        cp = pltpu.async_copy(posp_hbm.at[idx_v], rowsp_v, sem)
        cx.wait()
        cp.wait()
        pltpu.sync_copy(rowsx_v, selx_hbm.at[pl.ds(g * K, K)])
        pltpu.sync_copy(rowsp_v, selp_hbm.at[pl.ds(g * K, K)])
        pltpu.sync_copy(mask_v, mask_hbm.at[pl.ds(g * K, K)])


def _project_body(selx_ref, selp_ref, wx_ref, wp_ref, b_ref, mask_ref,
                  out_ref):
    h = (jnp.dot(selx_ref[...], wx_ref[...],
                 preferred_element_type=jnp.float32)
         + jnp.dot(selp_ref[...], wp_ref[...],
                   preferred_element_type=jnp.float32))
    out_ref[...] = (h + b_ref[...]) * mask_ref[...]


def kernel(x, pos, W, b, batch):
    f32 = jnp.float32
    x = x.astype(f32)
    pos = pos.astype(f32)
    W = W.astype(f32)
    posp = jnp.pad(pos, ((0, 0), (0, D_IN - 3)))
    wx = W[:D_IN, D_OUT - 1:D_OUT]
    wp = W[D_IN:D_IN + 3, D_OUT - 1:D_OUT]
    Wx = W[:D_IN, :]
    Wp = jnp.zeros((D_IN, D_OUT), f32).at[:3, :].set(W[D_IN:D_IN + 3, :])
    batch_p = jnp.full((NPAD, 1), B, jnp.int32)
    batch_p = batch_p.at[:N, 0].set(batch.astype(jnp.int32))

    nfull = (N - 1) // BLK

    scores, counts2d = pl.pallas_call(
        _score_count_body,
        grid=(NPAD // BLK,),
        in_specs=[
            pl.BlockSpec((BLK, D_IN), lambda i: (jnp.minimum(i, nfull), 0)),
            pl.BlockSpec((BLK, 3), lambda i: (jnp.minimum(i, nfull), 0)),
            pl.BlockSpec((D_IN, 1), lambda i: (0, 0)),
            pl.BlockSpec((3, 1), lambda i: (0, 0)),
            pl.BlockSpec((BLK, 1), lambda i: (i, 0)),
        ],
        out_specs=[
            pl.BlockSpec((BLK, 1), lambda i: (i, 0)),
            pl.BlockSpec((8, B), lambda i: (0, 0)),
        ],
        out_shape=[
            jax.ShapeDtypeStruct((NPAD, 1), f32),
            jax.ShapeDtypeStruct((8, B), jnp.int32),
        ],
    )(x, pos, wx, wp, batch_p)

    mesh = plsc.VectorSubcoreMesh(core_axis_name="c", subcore_axis_name="s")
    topk = functools.partial(
        pl.kernel,
        mesh=mesh,
        compiler_params=pltpu.CompilerParams(needs_layout_passes=False),
        out_type=[
            jax.ShapeDtypeStruct((B * K, D_IN), f32),
            jax.ShapeDtypeStruct((B * K, D_IN), f32),
            jax.ShapeDtypeStruct((B * K,), f32),
        ],
        scratch_types=[
            pltpu.VMEM((B + 16,), jnp.int32),
            pltpu.VMEM((28 * CH,), f32),
            pltpu.VMEM((K,), jnp.int32),
            pltpu.VMEM((K, D_IN), f32),
            pltpu.VMEM((K, D_IN), f32),
            pltpu.VMEM((K,), f32),
            pltpu.SemaphoreType.DMA,
        ],
    )(_topk_gather_body)
    selx, selp, maskv = topk(scores.reshape(NPAD), counts2d[0], x, posp)

    pooled2d = pl.pallas_call(
        _project_body,
        out_shape=jax.ShapeDtypeStruct((B * K, D_OUT), f32),
    )(selx, selp, Wx, Wp, b.astype(f32).reshape(1, D_OUT),
      maskv.reshape(B * K, 1))

    pooled = pooled2d.reshape(B, K * D_OUT)
    pos_out = jnp.zeros((B, 3), f32)
    batch_out = jnp.arange(B, dtype=jnp.int64)
    return (pooled, pos_out, batch_out)
